# Initial kernel scaffold; baseline (speedup 1.0000x reference)
import jax
import jax.numpy as jnp
from jax.experimental import pallas as pl


def kernel(x, edge_index, edge_attr, batch, W1, b1, W2, b2, lin1_W, lin1_b, lin2_W, lin2_b):
    raise NotImplementedError("write your pallas kernel here")


# speedup vs baseline: 5.5963x; 5.5963x over previous
import functools

import jax
import jax.numpy as jnp
from jax import lax
from jax.experimental import pallas as pl
from jax.experimental.pallas import tpu as pltpu
from jax.experimental.pallas import tpu_sc as plsc

_N = 10000
_NP = 10240
_E = 160000
_EP = 163840
_NC, _NS, _L = 2, 16, 16
_NW = _NC * _NS
_CH = 40
_C = 128
_RPT = _NP // _NS
_BR = 640
_BR2 = 2000
_G = 64


@functools.cache
def _get_mesh():
    return plsc.VectorSubcoreMesh(
        core_axis_name="c", subcore_axis_name="s",
        num_cores=_NC, num_subcores=_NS,
    )


@functools.cache
def _get_sc_deg():
    @functools.partial(
        pl.kernel,
        out_type=(
            jax.ShapeDtypeStruct((_NC, _NP), jnp.float32),
            jax.ShapeDtypeStruct((_NW, _CH, _C), jnp.float32),
        ),
        mesh=_get_mesh(),
        compiler_params=pltpu.CompilerParams(use_tc_tiling_on_sc=False),
        scratch_types=[
            pltpu.VMEM((_CH, _C), jnp.int32),
            pltpu.VMEM((_CH, _C), jnp.int32),
            pltpu.VMEM((_CH, _C), jnp.float32),
            pltpu.VMEM_SHARED((_NP,), jnp.float32),
            pltpu.SemaphoreType.DMA,
        ],
    )
    def _sc_deg(rowi, coli, ea, zer, degp, wp_out, rv, cv, wv, acc, sem):
        c = lax.axis_index("c")
        s = lax.axis_index("s")
        wid = c * _NS + s
        pltpu.sync_copy(
            zer.at[pl.ds(s * _RPT, _RPT)], acc.at[pl.ds(s * _RPT, _RPT)]
        )
        pltpu.sync_copy(rowi.at[wid], rv)
        pltpu.sync_copy(coli.at[wid], cv)
        pltpu.sync_copy(ea.at[wid], wv)

        def zero_self(j, carry):
            for l in range(_C // _L):
                sl = pl.ds(l * _L, _L)
                r = rv[j, sl]
                cc = cv[j, sl]
                w = wv[j, sl]
                wv[j, sl] = jnp.where(r == cc, 0.0, w)
            return carry

        lax.fori_loop(0, _CH, zero_self, 0)
        pltpu.sync_copy(wv, wp_out.at[wid])
        plsc.subcore_barrier()

        def scat(j, carry):
            pltpu.sync_copy(wv.at[j], acc.at[rv.at[j]], add=True)
            return carry

        lax.fori_loop(0, _CH, scat, 0)
        plsc.subcore_barrier()
        pltpu.sync_copy(
            acc.at[pl.ds(s * _RPT, _RPT)], degp.at[c, pl.ds(s * _RPT, _RPT)]
        )

    return _sc_deg


@functools.cache
def _make_spmm(width):
    nsub = width // _L

    @functools.partial(
        pl.kernel,
        out_type=jax.ShapeDtypeStruct((_NC, _NP, width), jnp.float32),
        mesh=_get_mesh(),
        compiler_params=pltpu.CompilerParams(use_tc_tiling_on_sc=False),
        scratch_types=[
            pltpu.VMEM((_CH, _C), jnp.int32),
            pltpu.VMEM((_CH, _C), jnp.int32),
            pltpu.VMEM((_CH, _C), jnp.float32),
            pltpu.VMEM((_C, width), jnp.float32),
            pltpu.VMEM_SHARED((_NP, width), jnp.float32),
            pltpu.SemaphoreType.DMA,
        ],
    )
    def _spmm(z, rowi, coli, wp, zer, out, rv, cv, wv, rows, acc, sem):
        c = lax.axis_index("c")
        s = lax.axis_index("s")
        wid = c * _NS + s
        pltpu.sync_copy(zer.at[pl.ds(s * _RPT, _RPT)], acc.at[pl.ds(s * _RPT, _RPT)])
        pltpu.sync_copy(rowi.at[wid], rv)
        pltpu.sync_copy(coli.at[wid], cv)
        pltpu.sync_copy(wp.at[wid], wv)
        plsc.subcore_barrier()

        def chunk(j, carry):
            pltpu.async_copy(z.at[rv.at[j]], rows, sem).wait()

            def group(g, inner):
                wvec = wv[j, pl.ds(g * _L, _L)]
                base = g * _L
                for lane in range(_L):
                    w = wvec[lane]
                    e = base + lane
                    for l in range(nsub):
                        sl = pl.ds(l * _L, _L)
                        rows[e, sl] = rows[e, sl] * w
                return inner

            lax.fori_loop(0, _C // _L, group, 0)
            pltpu.sync_copy(rows, acc.at[cv.at[j]], add=True)
            return carry

        lax.fori_loop(0, _CH, chunk, 0)
        plsc.subcore_barrier()
        pltpu.sync_copy(
            acc.at[pl.ds(s * _RPT, _RPT)], out.at[c, pl.ds(s * _RPT, _RPT)]
        )

    return _spmm


def _tc_dis(degp):
    def body(d_ref, o_ref):
        deg = d_ref[0] + d_ref[1]
        o_ref[...] = jnp.where(deg > 0, lax.rsqrt(deg), 0.0)

    return pl.pallas_call(
        body, out_shape=jax.ShapeDtypeStruct((80, 128), jnp.float32)
    )(degp)


def _tc_mm1(xp, W1r, Sv):
    def body(x_ref, w_ref, s_ref, *out_refs):
        a = jnp.dot(x_ref[...], w_ref[...], preferred_element_type=jnp.float32)
        for k in range(6):
            out_refs[k][...] = a[:, k * 64 : (k + 1) * 64]
        out_refs[6][...] = s_ref[...] * a[:, 5 * 64 : 6 * 64]

    return pl.pallas_call(
        body,
        grid=(_NP // _BR,),
        in_specs=[
            pl.BlockSpec((_BR, 256), lambda i: (i, 0)),
            pl.BlockSpec((256, 384), lambda i: (0, 0)),
            pl.BlockSpec((_BR, 1), lambda i: (i, 0)),
        ],
        out_specs=[pl.BlockSpec((_BR, 64), lambda i: (i, 0))] * 7,
        out_shape=[jax.ShapeDtypeStruct((_NP, 64), jnp.float32)] * 7,
    )(xp, W1r, Sv)


def _combine(width, A, Bp, P, Sv):
    def body(a_ref, bp_ref, p_ref, s_ref, b_ref, y_ref):
        S = s_ref[...]
        bv = a_ref[...] - 2.0 * S * (p_ref[0] + p_ref[1]) - bp_ref[...]
        b_ref[...] = bv
        y_ref[...] = S * bv

    return pl.pallas_call(
        body,
        grid=(_NP // _BR,),
        in_specs=[
            pl.BlockSpec((_BR, width), lambda i: (i, 0)),
            pl.BlockSpec((_BR, width), lambda i: (i, 0)),
            pl.BlockSpec((2, _BR, width), lambda i: (0, i, 0)),
            pl.BlockSpec((_BR, 1), lambda i: (i, 0)),
        ],
        out_specs=[
            pl.BlockSpec((_BR, width), lambda i: (i, 0)),
            pl.BlockSpec((_BR, width), lambda i: (i, 0)),
        ],
        out_shape=[
            jax.ShapeDtypeStruct((_NP, width), jnp.float32),
            jax.ShapeDtypeStruct((_NP, width), jnp.float32),
        ],
    )(A, Bp, P, Sv)


def _tc_final1(A0, P, Bp, Sv, bias1, W2r):
    def body(a_ref, bp_ref, p_ref, s_ref, bb_ref, w_ref, *out_refs):
        S = s_ref[...]
        v = a_ref[...] - S * (p_ref[0] + p_ref[1]) - bp_ref[...] + bb_ref[...]
        h = jnp.where(v >= 0, v, 0.01 * v)
        a2 = jnp.dot(h, w_ref[...], preferred_element_type=jnp.float32)
        for k in range(6):
            out_refs[k][...] = a2[:, k * 32 : (k + 1) * 32]
        out_refs[6][...] = S * a2[:, 5 * 32 : 6 * 32]

    return pl.pallas_call(
        body,
        grid=(_NP // _BR,),
        in_specs=[
            pl.BlockSpec((_BR, 64), lambda i: (i, 0)),
            pl.BlockSpec((_BR, 64), lambda i: (i, 0)),
            pl.BlockSpec((2, _BR, 64), lambda i: (0, i, 0)),
            pl.BlockSpec((_BR, 1), lambda i: (i, 0)),
            pl.BlockSpec((1, 64), lambda i: (0, 0)),
            pl.BlockSpec((64, 192), lambda i: (0, 0)),
        ],
        out_specs=[pl.BlockSpec((_BR, 32), lambda i: (i, 0))] * 7,
        out_shape=[jax.ShapeDtypeStruct((_NP, 32), jnp.float32)] * 7,
    )(A0, Bp, P, Sv, bias1, W2r)


def _tc_final2(A2, P, Bp, Sv, bias2, batch3, w1, c1, w2, c2):
    nblk = _N // _BR2

    def body(
        a_ref, bp_ref, p_ref, s_ref, bb_ref, bt_ref, w1_ref, c1_ref, w2_ref,
        c2_ref, x1_ref, x2_ref, acc_s, acc_c,
    ):
        i = pl.program_id(0)
        S = s_ref[...]
        v = a_ref[...] - S * (p_ref[0] + p_ref[1]) - bp_ref[...] + bb_ref[...]
        h = jnp.where(v >= 0, v, 0.01 * v)
        bt = bt_ref[0]
        ids = lax.broadcasted_iota(jnp.int32, (_G, _BR2), 0)
        oh = (ids == bt).astype(jnp.float32)
        sums = jnp.dot(oh, h, preferred_element_type=jnp.float32)
        cnt = jnp.sum(oh, axis=1, keepdims=True)

        @pl.when(i == 0)
        def _():
            acc_s[...] = jnp.zeros_like(acc_s)
            acc_c[...] = jnp.zeros_like(acc_c)

        acc_s[...] += sums
        acc_c[...] += cnt

        @pl.when(i == nblk - 1)
        def _():
            p = acc_s[...] / jnp.maximum(acc_c[...], 1.0)
            x1 = jnp.dot(p, w1_ref[...], preferred_element_type=jnp.float32)
            x1 = x1 + c1_ref[...]
            x1_ref[...] = x1
            x2_ref[...] = (
                jnp.dot(x1, w2_ref[...], preferred_element_type=jnp.float32)
                + c2_ref[...]
            )

    return pl.pallas_call(
        body,
        grid=(nblk,),
        in_specs=[
            pl.BlockSpec((_BR2, 32), lambda i: (i, 0)),
            pl.BlockSpec((_BR2, 32), lambda i: (i, 0)),
            pl.BlockSpec((2, _BR2, 32), lambda i: (0, i, 0)),
            pl.BlockSpec((_BR2, 1), lambda i: (i, 0)),
            pl.BlockSpec((1, 32), lambda i: (0, 0)),
            pl.BlockSpec((1, 1, _BR2), lambda i: (i, 0, 0)),
            pl.BlockSpec((32, 16), lambda i: (0, 0)),
            pl.BlockSpec((1, 16), lambda i: (0, 0)),
            pl.BlockSpec((16, 1), lambda i: (0, 0)),
            pl.BlockSpec((1, 1), lambda i: (0, 0)),
        ],
        out_specs=[
            pl.BlockSpec((_G, 16), lambda i: (0, 0)),
            pl.BlockSpec((_G, 1), lambda i: (0, 0)),
        ],
        out_shape=[
            jax.ShapeDtypeStruct((_G, 16), jnp.float32),
            jax.ShapeDtypeStruct((_G, 1), jnp.float32),
        ],
        scratch_shapes=[
            pltpu.VMEM((_G, 32), jnp.float32),
            pltpu.VMEM((_G, 1), jnp.float32),
        ],
    )(A2, Bp, P, Sv, bias2, batch3, w1, c1, w2, c2)


def kernel(x, edge_index, edge_attr, batch, W1, b1, W2, b2, lin1_W, lin1_b,
           lin2_W, lin2_b):
    f32 = jnp.float32
    row = edge_index[0]
    col = edge_index[1]
    pad_e = _EP - _E
    rowp = jnp.concatenate([row, jnp.zeros((pad_e,), jnp.int32)]).reshape(
        _NW, _CH, _C
    )
    colp = jnp.concatenate([col, jnp.zeros((pad_e,), jnp.int32)]).reshape(
        _NW, _CH, _C
    )
    eap = jnp.concatenate([edge_attr, jnp.zeros((pad_e,), f32)]).reshape(
        _NW, _CH, _C
    )
    zer1 = jnp.zeros((_NP,), f32)
    zer64 = jnp.zeros((_NP, 64), f32)
    zer32 = jnp.zeros((_NP, 32), f32)

    _sc_deg = _get_sc_deg()
    _spmm64 = _make_spmm(64)
    _spmm32 = _make_spmm(32)

    degp, wp = _sc_deg(rowp, colp, eap, zer1)
    dis = _tc_dis(degp.reshape(_NC, 80, 128))
    Sv = dis.reshape(_NP, 1)

    xp = jnp.concatenate([x, jnp.zeros((_NP - _N, 256), f32)])
    W1r = jnp.transpose(W1, (1, 0, 2)).reshape(256, 6 * 64)
    *A, y = _tc_mm1(xp, W1r, Sv)

    P = _spmm64(y, rowp, colp, wp, zer64)
    b4, y = _combine(64, A[4], zer64, P, Sv)
    P = _spmm64(y, rowp, colp, wp, zer64)
    b3, y = _combine(64, A[3], A[5], P, Sv)
    P = _spmm64(y, rowp, colp, wp, zer64)
    b2_, y = _combine(64, A[2], b4, P, Sv)
    P = _spmm64(y, rowp, colp, wp, zer64)
    b1_, y = _combine(64, A[1], b3, P, Sv)
    P = _spmm64(y, rowp, colp, wp, zer64)
    W2r = jnp.transpose(W2, (1, 0, 2)).reshape(64, 6 * 32)
    *A, y = _tc_final1(A[0], P, b2_, Sv, b1.reshape(1, 64), W2r)

    P = _spmm32(y, rowp, colp, wp, zer32)
    b4, y = _combine(32, A[4], zer32, P, Sv)
    P = _spmm32(y, rowp, colp, wp, zer32)
    b3, y = _combine(32, A[3], A[5], P, Sv)
    P = _spmm32(y, rowp, colp, wp, zer32)
    b2_, y = _combine(32, A[2], b4, P, Sv)
    P = _spmm32(y, rowp, colp, wp, zer32)
    b1_, y = _combine(32, A[1], b3, P, Sv)
    P = _spmm32(y, rowp, colp, wp, zer32)

    batch3 = batch.reshape(_N // _BR2, 1, _BR2)
    x1, x2 = _tc_final2(
        A[0], P, b2_, Sv, b2.reshape(1, 32), batch3, lin1_W,
        lin1_b.reshape(1, 16), lin2_W, lin2_b.reshape(1, 1),
    )
    return (x1, x2)


# speedup vs baseline: 8.2769x; 1.4790x over previous
import functools

import jax
import jax.numpy as jnp
from jax import lax
from jax.experimental import pallas as pl
from jax.experimental.pallas import tpu as pltpu
from jax.experimental.pallas import tpu_sc as plsc

_N = 10000
_NP = 10240
_E = 160000
_EP = 163840
_NC, _NS, _L = 2, 16, 16
_NW = _NC * _NS
_CH = 40
_C = 128
_RPT = _NP // _NS
_BR = 640
_BR2 = 2000
_G = 64


@functools.cache
def _get_mesh():
    return plsc.VectorSubcoreMesh(
        core_axis_name="c", subcore_axis_name="s",
        num_cores=_NC, num_subcores=_NS,
    )


@functools.cache
def _get_sc_deg():
    @functools.partial(
        pl.kernel,
        out_type=(
            jax.ShapeDtypeStruct((_NC, _NP), jnp.float32),
            jax.ShapeDtypeStruct((_NW, _CH, _C), jnp.float32),
        ),
        mesh=_get_mesh(),
        compiler_params=pltpu.CompilerParams(use_tc_tiling_on_sc=False),
        scratch_types=[
            pltpu.VMEM((_CH, _C), jnp.int32),
            pltpu.VMEM((_CH, _C), jnp.int32),
            pltpu.VMEM((_CH, _C), jnp.float32),
            pltpu.VMEM_SHARED((_NP,), jnp.float32),
            pltpu.SemaphoreType.DMA,
        ],
    )
    def _sc_deg(rowi, coli, ea, zer, degp, wp_out, rv, cv, wv, acc, sem):
        c = lax.axis_index("c")
        s = lax.axis_index("s")
        wid = c * _NS + s
        pltpu.sync_copy(
            zer.at[pl.ds(s * _RPT, _RPT)], acc.at[pl.ds(s * _RPT, _RPT)]
        )
        pltpu.sync_copy(rowi.at[wid], rv)
        pltpu.sync_copy(coli.at[wid], cv)
        pltpu.sync_copy(ea.at[wid], wv)

        def zero_self(j, carry):
            for l in range(_C // _L):
                sl = pl.ds(l * _L, _L)
                r = rv[j, sl]
                cc = cv[j, sl]
                w = wv[j, sl]
                wv[j, sl] = jnp.where(r == cc, 0.0, w)
            return carry

        lax.fori_loop(0, _CH, zero_self, 0)
        pltpu.sync_copy(wv, wp_out.at[wid])
        plsc.subcore_barrier()

        def scat(j, carry):
            pltpu.sync_copy(wv.at[j], acc.at[rv.at[j]], add=True)
            return carry

        lax.fori_loop(0, _CH, scat, 0)
        plsc.subcore_barrier()
        pltpu.sync_copy(
            acc.at[pl.ds(s * _RPT, _RPT)], degp.at[c, pl.ds(s * _RPT, _RPT)]
        )

    return _sc_deg


@functools.cache
def _make_spmm(width):
    nsub = width // _L

    @functools.partial(
        pl.kernel,
        out_type=jax.ShapeDtypeStruct((_NC, _NP, width), jnp.float32),
        mesh=_get_mesh(),
        compiler_params=pltpu.CompilerParams(use_tc_tiling_on_sc=False),
        scratch_types=[
            pltpu.VMEM((_CH, _C), jnp.int32),
            pltpu.VMEM((_CH, _C), jnp.int32),
            pltpu.VMEM((_CH, _C), jnp.float32),
            pltpu.VMEM((2, _C, width), jnp.float32),
            pltpu.VMEM((2, _C, width), jnp.float32),
            pltpu.VMEM_SHARED((_NP, width), jnp.float32),
            pltpu.SemaphoreType.DMA,
            pltpu.SemaphoreType.DMA,
            pltpu.SemaphoreType.DMA,
            pltpu.SemaphoreType.DMA,
        ],
    )
    def _spmm(z, rowi, coli, wp, zer, out, rv, cv, wv, gbuf, obuf, acc,
              gs0, gs1, ss0, ss1, ):
        c = lax.axis_index("c")
        s = lax.axis_index("s")
        wid = c * _NS + s
        gsem = (gs0, gs1)
        ssem = (ss0, ss1)
        pltpu.sync_copy(zer.at[pl.ds(s * _RPT, _RPT)], acc.at[pl.ds(s * _RPT, _RPT)])
        pltpu.sync_copy(rowi.at[wid], rv)
        pltpu.sync_copy(coli.at[wid], cv)
        pltpu.sync_copy(wp.at[wid], wv)
        plsc.subcore_barrier()

        pltpu.async_copy(z.at[rv.at[0]], gbuf.at[0], gsem[0])
        pltpu.async_copy(z.at[rv.at[1]], gbuf.at[1], gsem[1])

        def step(j0, carry):
            for b in range(2):
                j = j0 + b
                gb = gbuf.at[b]
                ob = obuf.at[b]
                pltpu.make_async_copy(z.at[rv.at[j]], gb, gsem[b]).wait()

                @pl.when(j0 >= 2)
                def _():
                    pltpu.make_async_copy(ob, acc.at[cv.at[j]], ssem[b]).wait()

                def group(g, inner):
                    wvec = wv[j, pl.ds(g * _L, _L)]
                    base = g * _L
                    for lane in range(_L):
                        w = wvec[lane]
                        e = base + lane
                        for l in range(nsub):
                            sl = pl.ds(l * _L, _L)
                            ob[e, sl] = gb[e, sl] * w
                    return inner

                lax.fori_loop(0, _C // _L, group, 0)
                pltpu.async_copy(ob, acc.at[cv.at[j]], ssem[b], add=True)

                @pl.when(j + 2 < _CH)
                def _():
                    pltpu.async_copy(z.at[rv.at[j + 2]], gb, gsem[b])
            return carry

        lax.fori_loop(0, _CH // 2, lambda i, cr: step(2 * i, cr), 0)
        pltpu.make_async_copy(
            obuf.at[0], acc.at[cv.at[_CH - 2]], ssem[0]
        ).wait()
        pltpu.make_async_copy(
            obuf.at[1], acc.at[cv.at[_CH - 1]], ssem[1]
        ).wait()
        plsc.subcore_barrier()
        pltpu.sync_copy(
            acc.at[pl.ds(s * _RPT, _RPT)], out.at[c, pl.ds(s * _RPT, _RPT)]
        )

    return _spmm


def _tc_dis(degp):
    def body(d_ref, o_ref):
        deg = d_ref[0] + d_ref[1]
        o_ref[...] = jnp.where(deg > 0, lax.rsqrt(deg), 0.0)

    return pl.pallas_call(
        body, out_shape=jax.ShapeDtypeStruct((80, 128), jnp.float32)
    )(degp)


def _tc_mm1(xp, W1r, Sv):
    def body(x_ref, w_ref, s_ref, *out_refs):
        a = jnp.dot(x_ref[...], w_ref[...], preferred_element_type=jnp.float32)
        for k in range(6):
            out_refs[k][...] = a[:, k * 64 : (k + 1) * 64]
        out_refs[6][...] = s_ref[...] * a[:, 5 * 64 : 6 * 64]

    return pl.pallas_call(
        body,
        grid=(_NP // _BR,),
        in_specs=[
            pl.BlockSpec((_BR, 256), lambda i: (i, 0)),
            pl.BlockSpec((256, 384), lambda i: (0, 0)),
            pl.BlockSpec((_BR, 1), lambda i: (i, 0)),
        ],
        out_specs=[pl.BlockSpec((_BR, 64), lambda i: (i, 0))] * 7,
        out_shape=[jax.ShapeDtypeStruct((_NP, 64), jnp.float32)] * 7,
    )(xp, W1r, Sv)


def _combine(width, A, Bp, P, Sv):
    def body(a_ref, bp_ref, p_ref, s_ref, b_ref, y_ref):
        S = s_ref[...]
        bv = a_ref[...] - 2.0 * S * (p_ref[0] + p_ref[1]) - bp_ref[...]
        b_ref[...] = bv
        y_ref[...] = S * bv

    return pl.pallas_call(
        body,
        grid=(_NP // _BR,),
        in_specs=[
            pl.BlockSpec((_BR, width), lambda i: (i, 0)),
            pl.BlockSpec((_BR, width), lambda i: (i, 0)),
            pl.BlockSpec((2, _BR, width), lambda i: (0, i, 0)),
            pl.BlockSpec((_BR, 1), lambda i: (i, 0)),
        ],
        out_specs=[
            pl.BlockSpec((_BR, width), lambda i: (i, 0)),
            pl.BlockSpec((_BR, width), lambda i: (i, 0)),
        ],
        out_shape=[
            jax.ShapeDtypeStruct((_NP, width), jnp.float32),
            jax.ShapeDtypeStruct((_NP, width), jnp.float32),
        ],
    )(A, Bp, P, Sv)


def _tc_final1(A0, P, Bp, Sv, bias1, W2r):
    def body(a_ref, bp_ref, p_ref, s_ref, bb_ref, w_ref, *out_refs):
        S = s_ref[...]
        v = a_ref[...] - S * (p_ref[0] + p_ref[1]) - bp_ref[...] + bb_ref[...]
        h = jnp.where(v >= 0, v, 0.01 * v)
        a2 = jnp.dot(h, w_ref[...], preferred_element_type=jnp.float32)
        for k in range(6):
            out_refs[k][...] = a2[:, k * 32 : (k + 1) * 32]
        out_refs[6][...] = S * a2[:, 5 * 32 : 6 * 32]

    return pl.pallas_call(
        body,
        grid=(_NP // _BR,),
        in_specs=[
            pl.BlockSpec((_BR, 64), lambda i: (i, 0)),
            pl.BlockSpec((_BR, 64), lambda i: (i, 0)),
            pl.BlockSpec((2, _BR, 64), lambda i: (0, i, 0)),
            pl.BlockSpec((_BR, 1), lambda i: (i, 0)),
            pl.BlockSpec((1, 64), lambda i: (0, 0)),
            pl.BlockSpec((64, 192), lambda i: (0, 0)),
        ],
        out_specs=[pl.BlockSpec((_BR, 32), lambda i: (i, 0))] * 7,
        out_shape=[jax.ShapeDtypeStruct((_NP, 32), jnp.float32)] * 7,
    )(A0, Bp, P, Sv, bias1, W2r)


def _tc_final2(A2, P, Bp, Sv, bias2, batch3, w1, c1, w2, c2):
    nblk = _N // _BR2

    def body(
        a_ref, bp_ref, p_ref, s_ref, bb_ref, bt_ref, w1_ref, c1_ref, w2_ref,
        c2_ref, x1_ref, x2_ref, acc_s, acc_c,
    ):
        i = pl.program_id(0)
        S = s_ref[...]
        v = a_ref[...] - S * (p_ref[0] + p_ref[1]) - bp_ref[...] + bb_ref[...]
        h = jnp.where(v >= 0, v, 0.01 * v)
        bt = bt_ref[0]
        ids = lax.broadcasted_iota(jnp.int32, (_G, _BR2), 0)
        oh = (ids == bt).astype(jnp.float32)
        sums = jnp.dot(oh, h, preferred_element_type=jnp.float32)
        cnt = jnp.sum(oh, axis=1, keepdims=True)

        @pl.when(i == 0)
        def _():
            acc_s[...] = jnp.zeros_like(acc_s)
            acc_c[...] = jnp.zeros_like(acc_c)

        acc_s[...] += sums
        acc_c[...] += cnt

        @pl.when(i == nblk - 1)
        def _():
            p = acc_s[...] / jnp.maximum(acc_c[...], 1.0)
            x1 = jnp.dot(p, w1_ref[...], preferred_element_type=jnp.float32)
            x1 = x1 + c1_ref[...]
            x1_ref[...] = x1
            x2_ref[...] = (
                jnp.dot(x1, w2_ref[...], preferred_element_type=jnp.float32)
                + c2_ref[...]
            )

    return pl.pallas_call(
        body,
        grid=(nblk,),
        in_specs=[
            pl.BlockSpec((_BR2, 32), lambda i: (i, 0)),
            pl.BlockSpec((_BR2, 32), lambda i: (i, 0)),
            pl.BlockSpec((2, _BR2, 32), lambda i: (0, i, 0)),
            pl.BlockSpec((_BR2, 1), lambda i: (i, 0)),
            pl.BlockSpec((1, 32), lambda i: (0, 0)),
            pl.BlockSpec((1, 1, _BR2), lambda i: (i, 0, 0)),
            pl.BlockSpec((32, 16), lambda i: (0, 0)),
            pl.BlockSpec((1, 16), lambda i: (0, 0)),
            pl.BlockSpec((16, 1), lambda i: (0, 0)),
            pl.BlockSpec((1, 1), lambda i: (0, 0)),
        ],
        out_specs=[
            pl.BlockSpec((_G, 16), lambda i: (0, 0)),
            pl.BlockSpec((_G, 1), lambda i: (0, 0)),
        ],
        out_shape=[
            jax.ShapeDtypeStruct((_G, 16), jnp.float32),
            jax.ShapeDtypeStruct((_G, 1), jnp.float32),
        ],
        scratch_shapes=[
            pltpu.VMEM((_G, 32), jnp.float32),
            pltpu.VMEM((_G, 1), jnp.float32),
        ],
    )(A2, Bp, P, Sv, bias2, batch3, w1, c1, w2, c2)


def kernel(x, edge_index, edge_attr, batch, W1, b1, W2, b2, lin1_W, lin1_b,
           lin2_W, lin2_b):
    f32 = jnp.float32
    row = edge_index[0]
    col = edge_index[1]
    pad_e = _EP - _E
    rowp = jnp.concatenate([row, jnp.zeros((pad_e,), jnp.int32)]).reshape(
        _NW, _CH, _C
    )
    colp = jnp.concatenate([col, jnp.zeros((pad_e,), jnp.int32)]).reshape(
        _NW, _CH, _C
    )
    eap = jnp.concatenate([edge_attr, jnp.zeros((pad_e,), f32)]).reshape(
        _NW, _CH, _C
    )
    zer1 = jnp.zeros((_NP,), f32)
    zer64 = jnp.zeros((_NP, 64), f32)
    zer32 = jnp.zeros((_NP, 32), f32)

    _sc_deg = _get_sc_deg()
    _spmm64 = _make_spmm(64)
    _spmm32 = _make_spmm(32)

    degp, wp = _sc_deg(rowp, colp, eap, zer1)
    dis = _tc_dis(degp.reshape(_NC, 80, 128))
    Sv = dis.reshape(_NP, 1)

    xp = jnp.concatenate([x, jnp.zeros((_NP - _N, 256), f32)])
    W1r = jnp.transpose(W1, (1, 0, 2)).reshape(256, 6 * 64)
    *A, y = _tc_mm1(xp, W1r, Sv)

    P = _spmm64(y, rowp, colp, wp, zer64)
    b4, y = _combine(64, A[4], zer64, P, Sv)
    P = _spmm64(y, rowp, colp, wp, zer64)
    b3, y = _combine(64, A[3], A[5], P, Sv)
    P = _spmm64(y, rowp, colp, wp, zer64)
    b2_, y = _combine(64, A[2], b4, P, Sv)
    P = _spmm64(y, rowp, colp, wp, zer64)
    b1_, y = _combine(64, A[1], b3, P, Sv)
    P = _spmm64(y, rowp, colp, wp, zer64)
    W2r = jnp.transpose(W2, (1, 0, 2)).reshape(64, 6 * 32)
    *A, y = _tc_final1(A[0], P, b2_, Sv, b1.reshape(1, 64), W2r)

    P = _spmm32(y, rowp, colp, wp, zer32)
    b4, y = _combine(32, A[4], zer32, P, Sv)
    P = _spmm32(y, rowp, colp, wp, zer32)
    b3, y = _combine(32, A[3], A[5], P, Sv)
    P = _spmm32(y, rowp, colp, wp, zer32)
    b2_, y = _combine(32, A[2], b4, P, Sv)
    P = _spmm32(y, rowp, colp, wp, zer32)
    b1_, y = _combine(32, A[1], b3, P, Sv)
    P = _spmm32(y, rowp, colp, wp, zer32)

    batch3 = batch.reshape(_N // _BR2, 1, _BR2)
    x1, x2 = _tc_final2(
        A[0], P, b2_, Sv, b2.reshape(1, 32), batch3, lin1_W,
        lin1_b.reshape(1, 16), lin2_W, lin2_b.reshape(1, 1),
    )
    return (x1, x2)
